# baseline (speedup 1.0000x reference)
import functools
import math

import jax
import jax.numpy as jnp
from jax import lax
from jax.experimental import pallas as pl
from jax.experimental.pallas import tpu as pltpu
from jax.experimental.pallas import tpu_sc as plsc

_LANES = 16


def _build_sc_embed(B, S, V, D, MAXS, NC, NS):
    NW = NC * NS
    pos_per_w = S // NW
    P = 8
    n_chunks = pos_per_w // P
    vecs_per_row = D // _LANES
    scale = math.sqrt(float(D))
    mesh = plsc.VectorSubcoreMesh(core_axis_name="c", subcore_axis_name="s")

    NBUF = 3

    @functools.partial(
        pl.kernel,
        out_type=jax.ShapeDtypeStruct((B, S, D), jnp.float32),
        mesh=mesh,
        scratch_types=[
            pltpu.VMEM((B, pos_per_w), jnp.int32),
            pltpu.VMEM((NBUF, B * P, D), jnp.float32),
            pltpu.VMEM((NBUF, P, D), jnp.float32),
        ] + [pltpu.SemaphoreType.DMA] * (2 * NBUF),
    )
    def sc_embed(x_hbm, table_hbm, pe_hbm, out_hbm, idx_v, rows_v, pe_v,
                 *sems):
        wid = lax.axis_index("s") * NC + lax.axis_index("c")
        pos_base = wid * pos_per_w
        gsems = sems[:NBUF]
        ssems = sems[NBUF:]

        for b in range(B):
            pltpu.sync_copy(x_hbm.at[b, pl.ds(pos_base, pos_per_w)],
                            idx_v.at[b])

        def issue_gather(c):
            buf = c % NBUF
            ops = []
            for b in range(B):
                ops.append(pltpu.async_copy(
                    table_hbm.at[idx_v.at[b, pl.ds(c * P, P)]],
                    rows_v.at[buf, pl.ds(b * P, P)],
                    gsems[buf]))
            ops.append(pltpu.async_copy(
                pe_hbm.at[0, pl.ds(pos_base + c * P, P)],
                pe_v.at[buf],
                gsems[buf]))
            return ops

        gathers = {}
        stores = {}
        for c0 in range(min(NBUF - 1, n_chunks)):
            gathers[c0] = issue_gather(c0)
        for c in range(n_chunks):
            buf = c % NBUF
            nxt = c + NBUF - 1
            if nxt < n_chunks:
                if c - 1 in stores:
                    for op in stores.pop(c - 1):
                        op.wait()
                gathers[nxt] = issue_gather(nxt)
            for op in gathers.pop(c):
                op.wait()

            def vec_body(i, buf=buf):
                p = i // vecs_per_row
                j = i - p * vecs_per_row
                sl = pl.ds(j * _LANES, _LANES)
                pv = pe_v[buf, p, sl]
                for b in range(B):
                    r = b * P + p
                    rows_v[buf, r, sl] = rows_v[buf, r, sl] * scale + pv

            plsc.parallel_loop(0, P * vecs_per_row, unroll=8)(vec_body)

            ops = []
            for b in range(B):
                ops.append(pltpu.async_copy(
                    rows_v.at[buf, pl.ds(b * P, P)],
                    out_hbm.at[b, pl.ds(pos_base + c * P, P)],
                    ssems[buf]))
            stores[c] = ops
        for c in sorted(stores):
            for op in stores[c]:
                op.wait()

    return sc_embed


@jax.jit
def kernel(x, table, pe):
    B, S = x.shape
    V, D = table.shape
    info = plsc.get_sparse_core_info()
    sc_embed = _build_sc_embed(B, S, V, D, pe.shape[1],
                               info.num_cores, info.num_subcores)
    x32 = x.astype(jnp.int32)
    return sc_embed(x32, table, pe)


# speedup vs baseline: 1.0026x; 1.0026x over previous
import functools
import math

import jax
import jax.numpy as jnp
from jax import lax
from jax.experimental import pallas as pl
from jax.experimental.pallas import tpu as pltpu
from jax.experimental.pallas import tpu_sc as plsc

_LANES = 16


def _build_sc_embed(B, S, V, D, MAXS, NC, NS):
    NW = NC * NS
    pos_per_w = S // NW
    P = 8
    n_chunks = pos_per_w // P
    vecs_per_row = D // _LANES
    scale = math.sqrt(float(D))
    mesh = plsc.VectorSubcoreMesh(core_axis_name="c", subcore_axis_name="s")

    NBUF = 3

    @functools.partial(
        pl.kernel,
        out_type=jax.ShapeDtypeStruct((B, S, D), jnp.float32),
        mesh=mesh,
        scratch_types=[
            pltpu.VMEM((B, pos_per_w), jnp.int32),
            pltpu.VMEM((NBUF, B * P, D), jnp.float32),
            pltpu.VMEM((NBUF, P, D), jnp.float32),
        ] + [pltpu.SemaphoreType.DMA] * (2 * NBUF),
    )
    def sc_embed(x_hbm, table_hbm, pe_hbm, out_hbm, idx_v, rows_v, pe_v,
                 *sems):
        wid = lax.axis_index("s") * NC + lax.axis_index("c")
        pos_base = wid * pos_per_w
        gsems = sems[:NBUF]
        ssems = sems[NBUF:]

        for b in range(B):
            pltpu.sync_copy(x_hbm.at[b, pl.ds(pos_base, pos_per_w)],
                            idx_v.at[b])

        def issue_gather(c):
            buf = c % NBUF
            ops = []
            for b in range(B):
                ops.append(pltpu.async_copy(
                    table_hbm.at[idx_v.at[b, pl.ds(c * P, P)]],
                    rows_v.at[buf, pl.ds(b * P, P)],
                    gsems[buf]))
            ops.append(pltpu.async_copy(
                pe_hbm.at[0, pl.ds(pos_base + c * P, P)],
                pe_v.at[buf],
                gsems[buf]))
            return ops

        gathers = {}
        stores = {}
        for c0 in range(min(NBUF - 1, n_chunks)):
            gathers[c0] = issue_gather(c0)
        for c in range(n_chunks):
            buf = c % NBUF
            nxt = c + NBUF - 1
            if nxt < n_chunks:
                if c - 1 in stores:
                    for op in stores.pop(c - 1):
                        op.wait()
                gathers[nxt] = issue_gather(nxt)
            for op in gathers.pop(c):
                op.wait()

            def vec_body(i, buf=buf):
                p = i // vecs_per_row
                j = i - p * vecs_per_row
                sl = pl.ds(j * _LANES, _LANES)
                pv = pe_v[buf, p, sl]
                for b in range(B):
                    r = b * P + p
                    rows_v[buf, r, sl] = rows_v[buf, r, sl] * scale + pv

            plsc.parallel_loop(0, P * vecs_per_row, unroll=4)(vec_body)

            ops = []
            for b in range(B):
                ops.append(pltpu.async_copy(
                    rows_v.at[buf, pl.ds(b * P, P)],
                    out_hbm.at[b, pl.ds(pos_base + c * P, P)],
                    ssems[buf]))
            stores[c] = ops
        for c in sorted(stores):
            for op in stores[c]:
                op.wait()

    return sc_embed


@jax.jit
def kernel(x, table, pe):
    B, S = x.shape
    V, D = table.shape
    info = plsc.get_sparse_core_info()
    sc_embed = _build_sc_embed(B, S, V, D, pe.shape[1],
                               info.num_cores, info.num_subcores)
    x32 = x.astype(jnp.int32)
    return sc_embed(x32, table, pe)


# speedup vs baseline: 1.0282x; 1.0255x over previous
import functools
import math

import jax
import jax.numpy as jnp
from jax import lax
from jax.experimental import pallas as pl
from jax.experimental.pallas import tpu as pltpu
from jax.experimental.pallas import tpu_sc as plsc

_LANES = 16


def _build_sc_embed(B, S, V, D, MAXS, NC, NS):
    NW = NC * NS
    pos_per_w = S // NW
    P = 8
    n_chunks = pos_per_w // P
    vecs_per_row = D // _LANES
    scale = math.sqrt(float(D))
    mesh = plsc.VectorSubcoreMesh(core_axis_name="c", subcore_axis_name="s")

    NBUF = 3

    @functools.partial(
        pl.kernel,
        out_type=jax.ShapeDtypeStruct((B, S, D), jnp.float32),
        mesh=mesh,
        scratch_types=[
            pltpu.VMEM((B, pos_per_w), jnp.int32),
            pltpu.VMEM((NBUF, B * P, D), jnp.float32),
            pltpu.VMEM((NBUF, P, D), jnp.float32),
        ] + [pltpu.SemaphoreType.DMA] * (2 * NBUF),
    )
    def sc_embed(x_hbm, table_hbm, pe_hbm, out_hbm, idx_v, rows_v, pe_v,
                 *sems):
        wid = lax.axis_index("s") * NC + lax.axis_index("c")
        pos_base = wid * pos_per_w
        gsems = sems[:NBUF]
        ssems = sems[NBUF:]

        pltpu.sync_copy(x_hbm.at[:, pl.ds(pos_base, pos_per_w)], idx_v)

        def issue_gather(c):
            buf = c % NBUF
            ops = []
            for b in range(B):
                ops.append(pltpu.async_copy(
                    table_hbm.at[idx_v.at[b, pl.ds(c * P, P)]],
                    rows_v.at[buf, pl.ds(b * P, P)],
                    gsems[buf]))
            ops.append(pltpu.async_copy(
                pe_hbm.at[0, pl.ds(pos_base + c * P, P)],
                pe_v.at[buf],
                gsems[buf]))
            return ops

        gathers = {}
        stores = {}
        for c0 in range(min(NBUF - 1, n_chunks)):
            gathers[c0] = issue_gather(c0)
        for c in range(n_chunks):
            buf = c % NBUF
            nxt = c + NBUF - 1
            if nxt < n_chunks:
                if c - 1 in stores:
                    for op in stores.pop(c - 1):
                        op.wait()
                gathers[nxt] = issue_gather(nxt)
            for op in gathers.pop(c):
                op.wait()

            def vec_body(i, buf=buf):
                p = i // vecs_per_row
                j = i - p * vecs_per_row
                sl = pl.ds(j * _LANES, _LANES)
                pv = pe_v[buf, p, sl]
                for b in range(B):
                    r = b * P + p
                    rows_v[buf, r, sl] = rows_v[buf, r, sl] * scale + pv

            plsc.parallel_loop(0, P * vecs_per_row, unroll=4)(vec_body)

            ops = []
            for b in range(B):
                ops.append(pltpu.async_copy(
                    rows_v.at[buf, pl.ds(b * P, P)],
                    out_hbm.at[b, pl.ds(pos_base + c * P, P)],
                    ssems[buf]))
            stores[c] = ops
        for c in sorted(stores):
            for op in stores[c]:
                op.wait()

    return sc_embed


@jax.jit
def kernel(x, table, pe):
    B, S = x.shape
    V, D = table.shape
    info = plsc.get_sparse_core_info()
    sc_embed = _build_sc_embed(B, S, V, D, pe.shape[1],
                               info.num_cores, info.num_subcores)
    x32 = x.astype(jnp.int32)
    return sc_embed(x32, table, pe)


# speedup vs baseline: 1.1173x; 1.0867x over previous
import functools
import math

import jax
import jax.numpy as jnp
from jax import lax
from jax.experimental import pallas as pl
from jax.experimental.pallas import tpu as pltpu
from jax.experimental.pallas import tpu_sc as plsc

_LANES = 16


def _build_sc_embed(B, S, V, D, MAXS, NC, NS):
    NW = NC * NS
    pos_per_w = S // NW
    P = 8
    n_chunks = pos_per_w // P
    vecs_per_row = D // _LANES
    scale = math.sqrt(float(D))
    mesh = plsc.VectorSubcoreMesh(core_axis_name="c", subcore_axis_name="s")

    NBUF = 2

    @functools.partial(
        pl.kernel,
        out_type=jax.ShapeDtypeStruct((B, S, D), jnp.float32),
        mesh=mesh,
        scratch_types=[
            pltpu.VMEM((B, pos_per_w), jnp.int32),
            pltpu.VMEM((NBUF, B * P, D), jnp.float32),
            pltpu.VMEM((NBUF, P, D), jnp.float32),
        ] + [pltpu.SemaphoreType.DMA] * (2 * NBUF),
    )
    def sc_embed(x_hbm, table_hbm, pe_hbm, out_hbm, idx_v, rows_v, pe_v,
                 *sems):
        wid = lax.axis_index("s") * NC + lax.axis_index("c")
        pos_base = wid * pos_per_w
        gsems = sems[:NBUF]
        ssems = sems[NBUF:]

        pltpu.sync_copy(x_hbm.at[:, pl.ds(pos_base, pos_per_w)], idx_v)

        def gather_descs(c, buf):
            descs = []
            for b in range(B):
                descs.append((
                    table_hbm.at[idx_v.at[b, pl.ds(c * P, P)]],
                    rows_v.at[buf, pl.ds(b * P, P)],
                    gsems[buf]))
            descs.append((
                pe_hbm.at[0, pl.ds(pos_base + c * P, P)],
                pe_v.at[buf],
                gsems[buf]))
            return descs

        def store_descs(c, buf):
            return [(rows_v.at[buf, pl.ds(b * P, P)],
                     out_hbm.at[b, pl.ds(pos_base + c * P, P)],
                     ssems[buf])
                    for b in range(B)]

        for buf0 in range(NBUF):
            for src, dst, sem in gather_descs(buf0, buf0):
                pltpu.async_copy(src, dst, sem)

        n_t = n_chunks // NBUF

        def round_body(t, carry):
            for k in range(NBUF):
                c = t * NBUF + k

                @pl.when(t >= 1)
                def _wait_prev_store(c=c, k=k):
                    for src, dst, sem in store_descs(c - NBUF, k):
                        pltpu.make_async_copy(src, dst, sem).wait()

                @pl.when(t < n_t - 1)
                def _issue_next(c=c, k=k):
                    for src, dst, sem in gather_descs(c + NBUF, k):
                        pltpu.async_copy(src, dst, sem)

                for src, dst, sem in gather_descs(c, k):
                    pltpu.make_async_copy(src, dst, sem).wait()

                def vec_body(i, k=k):
                    p = i // vecs_per_row
                    j = i - p * vecs_per_row
                    sl = pl.ds(j * _LANES, _LANES)
                    pv = pe_v[k, p, sl]
                    for b in range(B):
                        r = b * P + p
                        rows_v[k, r, sl] = rows_v[k, r, sl] * scale + pv

                plsc.parallel_loop(0, P * vecs_per_row, unroll=4)(vec_body)

                for src, dst, sem in store_descs(c, k):
                    pltpu.async_copy(src, dst, sem)
            return carry

        lax.fori_loop(0, n_t, round_body, 0)
        for k in range(NBUF):
            c = n_chunks - NBUF + k
            for src, dst, sem in store_descs(c, k):
                pltpu.make_async_copy(src, dst, sem).wait()

    return sc_embed


@jax.jit
def kernel(x, table, pe):
    B, S = x.shape
    V, D = table.shape
    info = plsc.get_sparse_core_info()
    sc_embed = _build_sc_embed(B, S, V, D, pe.shape[1],
                               info.num_cores, info.num_subcores)
    x32 = x.astype(jnp.int32)
    return sc_embed(x32, table, pe)
